# Initial kernel scaffold; baseline (speedup 1.0000x reference)
import jax
import jax.numpy as jnp
from jax.experimental import pallas as pl


def kernel(y0, y1, y2, t0, t1, t2, p_y):
    raise NotImplementedError("write your pallas kernel here")


# speedup vs baseline: 3.6385x; 3.6385x over previous
import numpy as np
import jax
import jax.numpy as jnp
from jax.experimental import pallas as pl
from jax.experimental.pallas import tpu as pltpu


def _bilinear_matrix(out_size: int, in_size: int) -> np.ndarray:
    W = np.zeros((out_size, in_size), dtype=np.float32)
    if out_size == 1:
        W[0, 0] = 1.0
        return W
    for i in range(out_size):
        src = i * (in_size - 1) / (out_size - 1)
        i0 = min(int(np.floor(src)), in_size - 1)
        i1 = min(i0 + 1, in_size - 1)
        w1 = src - i0
        W[i, i0] += 1.0 - w1
        W[i, i1] += w1
    return W


def _fused_body(y0_ref, y1_ref, y2_ref, t0_ref, t1_ref, t2_ref, p_ref,
                kh0_ref, w0_ref, kh1_ref, w1_ref,
                o0_ref, o1_ref, o2_ref, o3_ref):
    C, H0, W0 = y0_ref.shape[1:]
    _, H1, W1 = y1_ref.shape[1:]
    _, Ht, Wt = t0_ref.shape[1:]

    t0 = t0_ref[0].reshape(C * Ht, Wt).astype(jnp.bfloat16)
    tw0 = jnp.dot(t0, w0_ref[...], preferred_element_type=jnp.float32)
    interp0 = jnp.dot(kh0_ref[...], tw0.astype(jnp.bfloat16),
                      preferred_element_type=jnp.float32)
    s0 = jnp.sum(jnp.abs(y0_ref[0].reshape(C * H0, W0) - interp0))
    o0_ref[...] = jnp.broadcast_to(s0, o0_ref.shape)

    t1 = t1_ref[0].reshape(C * Ht, Wt).astype(jnp.bfloat16)
    tw1 = jnp.dot(t1, w1_ref[...], preferred_element_type=jnp.float32)
    interp1 = jnp.dot(kh1_ref[...], tw1.astype(jnp.bfloat16),
                      preferred_element_type=jnp.float32)
    s1 = jnp.sum(jnp.abs(y1_ref[0].reshape(C * H1, W1) - interp1))
    o1_ref[...] = jnp.broadcast_to(s1, o1_ref.shape)

    s2 = jnp.sum(jnp.abs(y2_ref[...] - t2_ref[...]))
    o2_ref[...] = jnp.broadcast_to(s2, o2_ref.shape)

    x = p_ref[0, 0]
    sig = 1.0 / (1.0 + jnp.exp(-x))
    s3 = jnp.sum(jnp.log(sig + 1e-9))
    o3_ref[...] = jnp.broadcast_to(s3, o3_ref.shape)


def kernel(y0, y1, y2, t0, t1, t2, p_y):
    N, C, H0, W0 = y0.shape
    _, _, H1, W1 = y1.shape
    _, _, H2, W2 = y2.shape
    _, _, Ht, Wt = t0.shape
    _, _, Hp, Wp = p_y.shape

    wh0 = _bilinear_matrix(H0, Ht)
    ww0 = _bilinear_matrix(W0, Wt)
    wh1 = _bilinear_matrix(H1, Ht)
    ww1 = _bilinear_matrix(W1, Wt)
    eye = np.eye(C, dtype=np.float32)
    kh0 = jnp.asarray(np.kron(eye, wh0), jnp.bfloat16)
    kh1 = jnp.asarray(np.kron(eye, wh1), jnp.bfloat16)
    w0t = jnp.asarray(ww0.T, jnp.bfloat16)
    w1t = jnp.asarray(ww1.T, jnp.bfloat16)

    out_sh = jax.ShapeDtypeStruct((N, 8, 128), jnp.float32)
    o0, o1, o2, o3 = pl.pallas_call(
        _fused_body,
        out_shape=(out_sh, out_sh, out_sh, out_sh),
        grid=(N,),
        in_specs=[
            pl.BlockSpec((1, C, H0, W0), lambda n: (n, 0, 0, 0)),
            pl.BlockSpec((1, C, H1, W1), lambda n: (n, 0, 0, 0)),
            pl.BlockSpec((1, C, H2, W2), lambda n: (n, 0, 0, 0)),
            pl.BlockSpec((1, C, Ht, Wt), lambda n: (n, 0, 0, 0)),
            pl.BlockSpec((1, C, Ht, Wt), lambda n: (n, 0, 0, 0)),
            pl.BlockSpec((1, C, Ht, Wt), lambda n: (n, 0, 0, 0)),
            pl.BlockSpec((1, 1, Hp, Wp), lambda n: (n, 0, 0, 0)),
            pl.BlockSpec((C * H0, C * Ht), lambda n: (0, 0)),
            pl.BlockSpec((Wt, W0), lambda n: (0, 0)),
            pl.BlockSpec((C * H1, C * Ht), lambda n: (0, 0)),
            pl.BlockSpec((Wt, W1), lambda n: (0, 0)),
        ],
        out_specs=(
            pl.BlockSpec((1, 8, 128), lambda n: (n, 0, 0)),
            pl.BlockSpec((1, 8, 128), lambda n: (n, 0, 0)),
            pl.BlockSpec((1, 8, 128), lambda n: (n, 0, 0)),
            pl.BlockSpec((1, 8, 128), lambda n: (n, 0, 0)),
        ),
        compiler_params=pltpu.CompilerParams(
            dimension_semantics=("parallel",),
            vmem_limit_bytes=64 * 1024 * 1024),
    )(y0, y1, y2, t0, t1, t2, p_y, kh0, w0t, kh1, w1t)

    s0 = jnp.sum(o0[:, 0, 0])
    s1 = jnp.sum(o1[:, 0, 0])
    s2 = jnp.sum(o2[:, 0, 0])
    s3 = jnp.sum(o3[:, 0, 0])

    n_levels = 3
    pyr_loss = jnp.zeros((1,), jnp.float32)
    pyr_loss = pyr_loss + (2.0 ** (n_levels - 2)) / N * s0
    pyr_loss = pyr_loss + (2.0 ** (n_levels - 3)) / N * s1
    rec_loss = s2 / N
    mean_logsig = s3 / float(N * 1 * Hp * Wp)
    adv_loss = -12.0 * 256.0 * 256.0 * mean_logsig
    loss = pyr_loss + rec_loss + adv_loss
    return rec_loss, pyr_loss, adv_loss, loss


# speedup vs baseline: 5.4075x; 1.4862x over previous
import numpy as np
import jax
import jax.numpy as jnp
from jax import lax
from jax.experimental import pallas as pl
from jax.experimental.pallas import tpu as pltpu

_GROUP = 2


def _bilinear_matrix(out_size: int, in_size: int) -> np.ndarray:
    W = np.zeros((out_size, in_size), dtype=np.float32)
    if out_size == 1:
        W[0, 0] = 1.0
        return W
    for i in range(out_size):
        src = i * (in_size - 1) / (out_size - 1)
        i0 = min(int(np.floor(src)), in_size - 1)
        i1 = min(i0 + 1, in_size - 1)
        w1 = src - i0
        W[i, i0] += 1.0 - w1
        W[i, i1] += w1
    return W


def _make_body(G, weights):
    w_pyr0, w_pyr1, w_rec, w_adv = weights

    def _fused_body(y0_ref, y1_ref, y2_ref, t0_ref, t1_ref, t2_ref, p_ref,
                    kh0_ref, w0_ref, kh1_ref, w1_ref, out_ref):
        B, C, H0, W0 = y0_ref.shape
        _, _, H1, W1 = y1_ref.shape
        _, _, Ht, Wt = t0_ref.shape
        _, _, Hp, Wp = p_ref.shape
        g = pl.program_id(0)

        t0 = t0_ref[...].reshape(B * C * Ht, Wt).astype(jnp.bfloat16)
        tw0 = jnp.dot(t0, w0_ref[...],
                      preferred_element_type=jnp.float32).astype(jnp.bfloat16)
        t1 = t1_ref[...].reshape(B * C * Ht, Wt).astype(jnp.bfloat16)
        tw1 = jnp.dot(t1, w1_ref[...],
                      preferred_element_type=jnp.float32).astype(jnp.bfloat16)
        s0 = jnp.float32(0.0)
        s1 = jnp.float32(0.0)
        for b in range(B):
            interp0 = jnp.dot(kh0_ref[...], tw0[b * C * Ht:(b + 1) * C * Ht],
                              preferred_element_type=jnp.float32)
            s0 += jnp.sum(jnp.abs(y0_ref[b].reshape(C * H0, W0) - interp0))
            interp1 = jnp.dot(kh1_ref[...], tw1[b * C * Ht:(b + 1) * C * Ht],
                              preferred_element_type=jnp.float32)
            s1 += jnp.sum(jnp.abs(y1_ref[b].reshape(C * H1, W1) - interp1))

        s2 = jnp.sum(jnp.abs(y2_ref[...] - t2_ref[...]))

        x = p_ref[...].reshape(B, Hp, Wp)
        sig = 1.0 / (1.0 + jnp.exp(-x))
        s3 = jnp.sum(jnp.log(sig + 1e-9))

        rows = lax.broadcasted_iota(jnp.int32, out_ref.shape, 0)
        contrib = jnp.where(rows == 0, s0,
                  jnp.where(rows == 1, s1,
                  jnp.where(rows == 2, s2,
                  jnp.where(rows == 3, s3, 0.0))))

        @pl.when(g == 0)
        def _():
            out_ref[...] = jnp.zeros_like(out_ref)

        out_ref[...] += contrib

        @pl.when(g == G - 1)
        def _():
            acc = out_ref[...]
            pyr = jnp.broadcast_to(w_pyr0 * acc[0:1] + w_pyr1 * acc[1:2],
                                   acc.shape)
            rec = jnp.broadcast_to(w_rec * acc[2:3], acc.shape)
            adv = jnp.broadcast_to(w_adv * acc[3:4], acc.shape)
            out_ref[...] = jnp.where(rows == 0, rec,
                           jnp.where(rows == 1, pyr,
                           jnp.where(rows == 2, adv,
                                     rec + pyr + adv)))

    return _fused_body


def kernel(y0, y1, y2, t0, t1, t2, p_y):
    N, C, H0, W0 = y0.shape
    _, _, H1, W1 = y1.shape
    _, _, H2, W2 = y2.shape
    _, _, Ht, Wt = t0.shape
    _, _, Hp, Wp = p_y.shape
    B = _GROUP
    G = N // B

    wh0 = _bilinear_matrix(H0, Ht)
    ww0 = _bilinear_matrix(W0, Wt)
    wh1 = _bilinear_matrix(H1, Ht)
    ww1 = _bilinear_matrix(W1, Wt)
    eye = np.eye(C, dtype=np.float32)
    kh0 = jnp.asarray(np.kron(eye, wh0), jnp.bfloat16)
    kh1 = jnp.asarray(np.kron(eye, wh1), jnp.bfloat16)
    w0t = jnp.asarray(ww0.T, jnp.bfloat16)
    w1t = jnp.asarray(ww1.T, jnp.bfloat16)

    n_levels = 3
    weights = ((2.0 ** (n_levels - 2)) / N,
               (2.0 ** (n_levels - 3)) / N,
               1.0 / N,
               -12.0 * 256.0 * 256.0 / float(N * Hp * Wp))

    out = pl.pallas_call(
        _make_body(G, weights),
        out_shape=jax.ShapeDtypeStruct((8, 128), jnp.float32),
        grid=(G,),
        in_specs=[
            pl.BlockSpec((B, C, H0, W0), lambda g: (g, 0, 0, 0)),
            pl.BlockSpec((B, C, H1, W1), lambda g: (g, 0, 0, 0)),
            pl.BlockSpec((B, C, H2, W2), lambda g: (g, 0, 0, 0)),
            pl.BlockSpec((B, C, Ht, Wt), lambda g: (g, 0, 0, 0)),
            pl.BlockSpec((B, C, Ht, Wt), lambda g: (g, 0, 0, 0)),
            pl.BlockSpec((B, C, Ht, Wt), lambda g: (g, 0, 0, 0)),
            pl.BlockSpec((B, 1, Hp, Wp), lambda g: (g, 0, 0, 0)),
            pl.BlockSpec((C * H0, C * Ht), lambda g: (0, 0)),
            pl.BlockSpec((Wt, W0), lambda g: (0, 0)),
            pl.BlockSpec((C * H1, C * Ht), lambda g: (0, 0)),
            pl.BlockSpec((Wt, W1), lambda g: (0, 0)),
        ],
        out_specs=pl.BlockSpec((8, 128), lambda g: (0, 0)),
        compiler_params=pltpu.CompilerParams(
            dimension_semantics=("arbitrary",),
            vmem_limit_bytes=64 * 1024 * 1024),
    )(y0, y1, y2, t0, t1, t2, p_y, kh0, w0t, kh1, w1t)

    rec_loss = out[0, 0]
    pyr_loss = out[1, 0:1]
    adv_loss = out[2, 0]
    loss = out[3, 0:1]
    return rec_loss, pyr_loss, adv_loss, loss
